# Initial kernel scaffold; baseline (speedup 1.0000x reference)
import jax
import jax.numpy as jnp
from jax.experimental import pallas as pl


def kernel(x, edge_index, W_embed, W_msg, b_msg, W_upd, U_upd, b_upd, W_out):
    raise NotImplementedError("write your pallas kernel here")


# speedup vs baseline: 2.8095x; 2.8095x over previous
import functools

import jax
import jax.numpy as jnp
from jax import lax
from jax.experimental import pallas as pl
from jax.experimental.pallas import tpu as pltpu
from jax.experimental.pallas import tpu_sc as plsc

N = 10000
E = 320000
H = 128
ITERS = 3
NC = 2
NS = 16
CHUNK = 128
N_PAD = 10240
BLK = 1280
GRID = N_PAD // BLK
NHALF = N_PAD // NC
ACC_ROWS = NHALF + CHUNK
ZERO_ROWS_PER_TILE = ACC_ROWS // NS
WB_ROWS_PER_TILE = NHALF // NS
N_CHUNKS = -(-E // (NS * CHUNK))
E_TILE = N_CHUNKS * CHUNK
E_PAD = E_TILE * NS


def _weights_body(wmsg_ref, wupd_ref, bmsg_ref, wmu_ref, bw_ref):
    wmu_ref[...] = jnp.dot(wmsg_ref[...], wupd_ref[...],
                           preferred_element_type=jnp.float32)
    bw_ref[...] = jnp.dot(bmsg_ref[...], wupd_ref[...],
                          preferred_element_type=jnp.float32)


_weights_prep = pl.pallas_call(
    _weights_body,
    out_shape=[jax.ShapeDtypeStruct((H, H), jnp.float32),
               jax.ShapeDtypeStruct((1, H), jnp.float32)],
)


def _embed_body(x_ref, we_ref, h_ref):
    h_ref[...] = jnp.dot(x_ref[...], we_ref[...],
                         preferred_element_type=jnp.float32)


_embed = pl.pallas_call(
    _embed_body,
    grid=(GRID,),
    in_specs=[pl.BlockSpec((BLK, H), lambda i: (i, 0)),
              pl.BlockSpec((H, H), lambda i: (0, 0))],
    out_specs=pl.BlockSpec((BLK, H), lambda i: (i, 0)),
    out_shape=jax.ShapeDtypeStruct((N_PAD, H), jnp.float32),
)


def _update_body(g_ref, deg_ref, h_ref, wmu_ref, uupd_ref,
                 bw_ref, bupd_ref, hn_ref):
    deg = deg_ref[:, 0:1]
    t = (jnp.dot(g_ref[...], wmu_ref[...], preferred_element_type=jnp.float32)
         + jnp.dot(h_ref[...], uupd_ref[...],
                   preferred_element_type=jnp.float32)
         + deg * bw_ref[...] + bupd_ref[...])
    row = (pl.program_id(0) * BLK
           + lax.broadcasted_iota(jnp.int32, (BLK, 1), 0))
    hn_ref[...] = jnp.where(row < N, jnp.tanh(t), 0.0)


_update = pl.pallas_call(
    _update_body,
    grid=(GRID,),
    in_specs=[pl.BlockSpec((BLK, H), lambda i: (i, 0)),
              pl.BlockSpec((BLK, 16), lambda i: (i, 0)),
              pl.BlockSpec((BLK, H), lambda i: (i, 0)),
              pl.BlockSpec((H, H), lambda i: (0, 0)),
              pl.BlockSpec((H, H), lambda i: (0, 0)),
              pl.BlockSpec((1, H), lambda i: (0, 0)),
              pl.BlockSpec((1, H), lambda i: (0, 0))],
    out_specs=pl.BlockSpec((BLK, H), lambda i: (i, 0)),
    out_shape=jax.ShapeDtypeStruct((N_PAD, H), jnp.float32),
)


def _readout_body(h_ref, wout_ref, o_ref):
    s = jnp.sum(h_ref[...], axis=0, keepdims=True)
    o_ref[...] = jnp.dot(s, wout_ref[...], preferred_element_type=jnp.float32)


_readout = pl.pallas_call(
    _readout_body,
    out_shape=jax.ShapeDtypeStruct((1, H), jnp.float32),
)


_sc_mesh = plsc.VectorSubcoreMesh(core_axis_name="c", subcore_axis_name="s")


def _zero_shared(zrow_hbm, stage_v, shared, r0):
    pltpu.sync_copy(zrow_hbm, stage_v)
    full, rem = divmod(ZERO_ROWS_PER_TILE, CHUNK)
    for k in range(full):
        pltpu.sync_copy(stage_v, shared.at[pl.ds(r0 + k * CHUNK, CHUNK)])
    if rem:
        pltpu.sync_copy(stage_v.at[pl.ds(0, rem)],
                        shared.at[pl.ds(r0 + full * CHUNK, rem)])


def _writeback(shared, stage_v, out_hbm, row0, r0):
    full, rem = divmod(WB_ROWS_PER_TILE, CHUNK)
    for k in range(full):
        pltpu.sync_copy(shared.at[pl.ds(r0 + k * CHUNK, CHUNK)], stage_v)
        pltpu.sync_copy(stage_v, out_hbm.at[pl.ds(row0 + r0 + k * CHUNK,
                                                  CHUNK)])
    if rem:
        pltpu.sync_copy(shared.at[pl.ds(r0 + full * CHUNK, rem)],
                        stage_v.at[pl.ds(0, rem)])
        pltpu.sync_copy(stage_v.at[pl.ds(0, rem)],
                        out_hbm.at[pl.ds(row0 + r0 + full * CHUNK, rem)])


def _sc_deg_body(h_hbm, src_hbm, dst_hbm, zh_hbm, z16_hbm, ones_hbm,
                 g_hbm, deg_hbm,
                 src_v, dst_v, rows_v, d16_v, ones_v, g_sh, deg_sh, sem):
    c = lax.axis_index("c")
    s = lax.axis_index("s")
    zr0 = s * ZERO_ROWS_PER_TILE
    _zero_shared(zh_hbm, rows_v, g_sh, zr0)
    _zero_shared(z16_hbm, d16_v, deg_sh, zr0)
    pltpu.sync_copy(ones_hbm, ones_v)
    plsc.subcore_barrier()
    ebase = s * E_TILE
    dbase = c * E_PAD + ebase

    def step(j, carry):
        pltpu.sync_copy(src_hbm.at[pl.ds(ebase + j * CHUNK, CHUNK)], src_v)
        pltpu.sync_copy(dst_hbm.at[pl.ds(dbase + j * CHUNK, CHUNK)], dst_v)
        pltpu.async_copy(h_hbm.at[src_v], rows_v, sem).wait()
        pltpu.sync_copy(rows_v, g_sh.at[dst_v], add=True)
        pltpu.sync_copy(ones_v, deg_sh.at[dst_v], add=True)
        return carry

    lax.fori_loop(0, N_CHUNKS, step, 0)
    plsc.subcore_barrier()
    wr0 = s * WB_ROWS_PER_TILE
    _writeback(g_sh, rows_v, g_hbm, c * NHALF, wr0)
    _writeback(deg_sh, d16_v, deg_hbm, c * NHALF, wr0)


_sc_pass_deg = functools.partial(
    pl.kernel,
    out_type=[jax.ShapeDtypeStruct((N_PAD, H), jnp.float32),
              jax.ShapeDtypeStruct((N_PAD, 16), jnp.float32)],
    mesh=_sc_mesh,
    scratch_types=[
        pltpu.VMEM((CHUNK,), jnp.int32),
        pltpu.VMEM((CHUNK,), jnp.int32),
        pltpu.VMEM((CHUNK, H), jnp.float32),
        pltpu.VMEM((CHUNK, 16), jnp.float32),
        pltpu.VMEM((CHUNK, 16), jnp.float32),
        pltpu.VMEM_SHARED((ACC_ROWS, H), jnp.float32),
        pltpu.VMEM_SHARED((ACC_ROWS, 16), jnp.float32),
        pltpu.SemaphoreType.DMA,
    ],
)(_sc_deg_body)


def _sc_body(h_hbm, src_hbm, dst_hbm, zh_hbm,
             g_hbm,
             src_v, dst_v, rows_v, g_sh, sem):
    c = lax.axis_index("c")
    s = lax.axis_index("s")
    zr0 = s * ZERO_ROWS_PER_TILE
    _zero_shared(zh_hbm, rows_v, g_sh, zr0)
    plsc.subcore_barrier()
    ebase = s * E_TILE
    dbase = c * E_PAD + ebase

    def step(j, carry):
        pltpu.sync_copy(src_hbm.at[pl.ds(ebase + j * CHUNK, CHUNK)], src_v)
        pltpu.sync_copy(dst_hbm.at[pl.ds(dbase + j * CHUNK, CHUNK)], dst_v)
        pltpu.async_copy(h_hbm.at[src_v], rows_v, sem).wait()
        pltpu.sync_copy(rows_v, g_sh.at[dst_v], add=True)
        return carry

    lax.fori_loop(0, N_CHUNKS, step, 0)
    plsc.subcore_barrier()
    wr0 = s * WB_ROWS_PER_TILE
    _writeback(g_sh, rows_v, g_hbm, c * NHALF, wr0)


_sc_pass = functools.partial(
    pl.kernel,
    out_type=[jax.ShapeDtypeStruct((N_PAD, H), jnp.float32)],
    mesh=_sc_mesh,
    scratch_types=[
        pltpu.VMEM((CHUNK,), jnp.int32),
        pltpu.VMEM((CHUNK,), jnp.int32),
        pltpu.VMEM((CHUNK, H), jnp.float32),
        pltpu.VMEM_SHARED((ACC_ROWS, H), jnp.float32),
        pltpu.SemaphoreType.DMA,
    ],
)(_sc_body)


@jax.jit
def _run(x, edge_index, W_embed, W_msg, b_msg, W_upd, U_upd, b_upd, W_out):
    xp = jnp.zeros((N_PAD, H), jnp.float32).at[:N].set(x)
    pad = E_PAD - E
    src_p = jnp.concatenate([edge_index[0], jnp.zeros((pad,), jnp.int32)])
    d = jnp.concatenate([edge_index[1],
                         jnp.full((pad,), -1, jnp.int32)])
    dummy = NHALF + (jnp.arange(E_PAD, dtype=jnp.int32) % CHUNK)
    dst_c0 = jnp.where((d >= 0) & (d < NHALF), d, dummy)
    dst_c1 = jnp.where(d >= NHALF, d - NHALF, dummy)
    dst2 = jnp.concatenate([dst_c0, dst_c1])
    zh = jnp.zeros((CHUNK, H), jnp.float32)
    z16 = jnp.zeros((CHUNK, 16), jnp.float32)
    ones = jnp.ones((CHUNK, 16), jnp.float32)

    wmu, bw = _weights_prep(W_msg, W_upd, b_msg.reshape(1, H))
    h = _embed(xp, W_embed)
    deg = jnp.zeros((N_PAD, 16), jnp.float32)
    for it in range(ITERS):
        (g,) = _sc_pass(h, src_p, dst2, zh)
        h = _update(g, deg, h, wmu, U_upd, bw, b_upd.reshape(1, H))
    out = _readout(h, W_out)
    return out.reshape(H)


def kernel(x, edge_index, W_embed, W_msg, b_msg, W_upd, U_upd, b_upd, W_out):
    return _run(x, edge_index, W_embed, W_msg, b_msg, W_upd, U_upd, b_upd,
                W_out)
